# baseline (speedup 1.0000x reference)
import functools

import jax
import jax.numpy as jnp
from jax import lax
from jax.experimental import pallas as pl
from jax.experimental.pallas import tpu as pltpu
from jax.experimental.pallas import tpu_sc as plsc

N = 50000
E = 1_600_000
NC, NS, L = 2, 16, 16
NW = NC * NS
NPAD = 53248
SEG = NPAD // NS
B = 1024
TBLK = (E + B - 1) // B
ELAST = E - B
EDUP = (TBLK - 1) * B
R2C = NPAD // NW
W2C = R2C * 3

_mesh = plsc.VectorSubcoreMesh(core_axis_name="c", subcore_axis_name="s")
_params = pltpu.CompilerParams(needs_layout_passes=False)

_GDN = lax.GatherDimensionNumbers(
    offset_dims=(), collapsed_slice_dims=(0,), start_index_map=(0,))


def _take16(x, idx):
    return lax.gather(x, idx[:, None], _GDN, slice_sizes=(1,),
                      mode=lax.GatherScatterMode.PROMISE_IN_BOUNDS)

_NI = 3
_NF = 7


@functools.partial(
    pl.kernel,
    out_type=jax.ShapeDtypeStruct((NC * 3 * NPAD,), jnp.float32),
    mesh=_mesh,
    compiler_params=_params,
    scratch_types=(
        [pltpu.VMEM_SHARED((NPAD,), jnp.float32) for _ in range(6)]
        + [pltpu.VMEM((2 * B,), jnp.int32) for _ in range(2 * _NI)]
        + [pltpu.VMEM((3 * B,), jnp.float32) for _ in range(2)]
        + [pltpu.VMEM((B,), jnp.float32) for _ in range(6)]
        + [pltpu.VMEM((2 * B,), jnp.float32) for _ in range(12)]
        + [pltpu.SemaphoreType.DMA for _ in range(5)]
    ),
)
def _edge_kernel(dip_h, zeros_h, idx_h, vec_h, out_h, *refs):
    DS = refs[0:3]
    AS = refs[3:6]
    RAW = (refs[6], refs[7])
    IDX = (refs[8], refs[9])
    IDXS = (refs[10], refs[11])
    VRAW = (refs[12], refs[13])
    VX = (refs[14], refs[15])
    VY = (refs[16], refs[17])
    VZ = (refs[18], refs[19])
    D = tuple((refs[20 + 2 * c], refs[21 + 2 * c]) for c in range(3))
    C = tuple((refs[26 + 2 * c], refs[27 + 2 * c]) for c in range(3))
    gsem0, gsem1, ssem0, ssem1, lsem = refs[32:]
    gsem = (gsem0, gsem1)
    ssem = (ssem0, ssem1)

    cid = lax.axis_index("c")
    sid = lax.axis_index("s")
    s0 = pl.multiple_of(sid * SEG, 128)
    for c in range(3):
        pltpu.sync_copy(dip_h.at[pl.ds(pl.multiple_of(c * NPAD + s0, 128), SEG)],
                        DS[c].at[pl.ds(s0, SEG)])
        pltpu.sync_copy(zeros_h.at[pl.ds(s0, SEG)], AS[c].at[pl.ds(s0, SEG)])
    plsc.subcore_barrier()

    wid = sid * NC + cid
    blo = (wid * TBLK) // NW
    bhi = ((wid + 1) * TBLK) // NW

    iota = lax.broadcasted_iota(jnp.int32, (L,), 0)
    p_ev = (iota * 2) % L
    p_od = p_ev + 1
    half = iota < (L // 2)
    P3 = tuple((iota * 3 + k) % L for k in range(3))
    LIMA = (6, 5, 5)
    LIMB = (11, 11, 10)

    def eoff(b):
        return pl.multiple_of(
            jnp.where(b >= TBLK - 1, ELAST, b * B).astype(jnp.int32), 128)

    def fire_linear(b, p):
        e0 = eoff(b)
        pltpu.async_copy(idx_h.at[pl.ds(pl.multiple_of(e0 * 2, 128), 2 * B)],
                         RAW[p], lsem)
        pltpu.async_copy(vec_h.at[pl.ds(pl.multiple_of(e0 * 3, 128), 3 * B)],
                         VRAW[p], lsem)

    def drain_linear(b, p):
        e0 = eoff(b)
        pltpu.make_async_copy(idx_h.at[pl.ds(pl.multiple_of(e0 * 2, 128), 2 * B)],
                              RAW[p], lsem).wait()
        pltpu.make_async_copy(vec_h.at[pl.ds(pl.multiple_of(e0 * 3, 128), 3 * B)],
                              VRAW[p], lsem).wait()

    def deint_idx(p):
        raw, idx = RAW[p], IDX[p]

        def dop(m, carry):
            a = raw[pl.ds(m * 2 * L, L)]
            b2 = raw[pl.ds(m * 2 * L + L, L)]
            ga = _take16(a, p_ev)
            gb = _take16(b2, p_ev)
            idx[pl.ds(m * L, L)] = jnp.where(half, ga, gb)
            ga = _take16(a, p_od)
            gb = _take16(b2, p_od)
            idx[pl.ds(B + m * L, L)] = jnp.where(half, ga, gb)
            return carry

        lax.fori_loop(0, B // L, dop, 0)

    def deint_vec(p):
        raw = VRAW[p]
        dsts = (VX[p], VY[p], VZ[p])

        def dop(m, carry):
            a = raw[pl.ds(m * 3 * L, L)]
            b2 = raw[pl.ds(m * 3 * L + L, L)]
            c2 = raw[pl.ds(m * 3 * L + 2 * L, L)]
            w = pl.ds(m * L, L)
            for k in range(3):
                ga = _take16(a, P3[k])
                gb = _take16(b2, P3[k])
                gc = _take16(c2, P3[k])
                dsts[k][w] = jnp.where(iota < LIMA[k], ga,
                                       jnp.where(iota < LIMB[k], gb, gc))
            return carry

        lax.fori_loop(0, B // L, dop, 0)

    def copy_idx(p):
        def cop(m, carry):
            w = pl.ds(m * L, L)
            IDXS[p][w] = IDX[p][w]
            return carry
        lax.fori_loop(0, 2 * B // L, cop, 0)

    def fire_gathers(p):
        for c in range(3):
            pltpu.async_copy(DS[c].at[IDX[p]], D[c][p], gsem[p])

    def drain_gathers(p):
        for c in range(3):
            pltpu.make_async_copy(DS[c].at[IDX[p]], D[c][p], gsem[p]).wait()

    def fire_scatters(p):
        for c in range(3):
            pltpu.async_copy(C[c][p], AS[c].at[IDXS[p]], ssem[p], add=True)

    def drain_scatters(p):
        for c in range(3):
            pltpu.make_async_copy(C[c][p], AS[c].at[IDXS[p]], ssem[p]).wait()

    def compute(b, p):
        vx_v, vy_v, vz_v = VX[p], VY[p], VZ[p]
        dx_v, dy_v, dz_v = D[0][p], D[1][p], D[2][p]
        cx_v, cy_v, cz_v = C[0][p], C[1][p], C[2][p]
        e0 = eoff(b)
        live = b < bhi
        islast = e0 == ELAST

        def vop(m, c3):
            wi = pl.ds(m * L, L)
            wj = pl.ds(B + m * L, L)
            vx = vx_v[wi]
            vy = vy_v[wi]
            vz = vz_v[wi]
            r2 = vx * vx + vy * vy + vz * vz
            bits = lax.bitcast_convert_type(r2, jnp.int32)
            y = lax.bitcast_convert_type(
                jnp.int32(0x5F3759DF) - (bits >> 1), jnp.float32)
            y = y * (1.5 - 0.5 * r2 * y * y)
            y = y * (1.5 - 0.5 * r2 * y * y)
            y = y * (1.5 - 0.5 * r2 * y * y)
            y2 = y * y
            sh = 0.5 * (y2 * y)
            th = 3.0 * (y2 * sh)
            keep = live & (~islast | ((e0 + m * L + iota) >= EDUP))
            sh = jnp.where(keep, sh, 0.0)
            th = jnp.where(keep, th, 0.0)
            djx = dx_v[wj]
            djy = dy_v[wj]
            djz = dz_v[wj]
            aj = (djx * vx + djy * vy + djz * vz) * th
            cx_v[wi] = djx * sh - vx * aj
            cy_v[wi] = djy * sh - vy * aj
            cz_v[wi] = djz * sh - vz * aj
            dix = dx_v[wi]
            diy = dy_v[wi]
            diz = dz_v[wi]
            ai = (dix * vx + diy * vy + diz * vz) * th
            cx_v[wj] = dix * sh - vx * ai
            cy_v[wj] = diy * sh - vy * ai
            cz_v[wj] = diz * sh - vz * ai
            return c3

        lax.fori_loop(0, B // L, vop, 0)

    fire_linear(blo, 0)
    drain_linear(blo, 0)
    deint_idx(0)
    fire_gathers(0)

    def superblock(t, carry):
        for p in (0, 1):
            b = blo + 2 * t + p
            drain_gathers(p)
            copy_idx(p)
            fire_linear(b + 1, p ^ 1)
            deint_vec(p)
            compute(b, p)
            if p == 0:
                @pl.when(t > 0)
                def _():
                    drain_scatters(1)
            else:
                drain_scatters(0)
            fire_scatters(p)
            drain_linear(b + 1, p ^ 1)
            deint_idx(p ^ 1)
            fire_gathers(p ^ 1)
        return carry

    nsup = (bhi - blo + 1) // 2
    lax.fori_loop(0, nsup, superblock, 0)
    drain_scatters(1)
    drain_gathers(0)
    plsc.subcore_barrier()
    for c in range(3):
        o = pl.multiple_of((cid * 3 + c) * NPAD + s0, 128)
        pltpu.sync_copy(AS[c].at[pl.ds(s0, SEG)], out_h.at[pl.ds(o, SEG)])


@functools.partial(
    pl.kernel,
    out_type=jax.ShapeDtypeStruct((NPAD * 3,), jnp.float32),
    mesh=_mesh,
    compiler_params=_params,
    scratch_types=(
        [pltpu.VMEM((R2C,), jnp.float32) for _ in range(6)]
        + [pltpu.VMEM((W2C,), jnp.float32)]
    ),
)
def _combine_kernel(p_h, out_h, p0x, p0y, p0z, p1x, p1y, p1z, stage):
    cid = lax.axis_index("c")
    sid = lax.axis_index("s")
    wid = sid * NC + cid
    r0 = pl.multiple_of(wid * R2C, 128)
    bufs = (p0x, p0y, p0z, p1x, p1y, p1z)
    for g in range(NC):
        for c in range(3):
            o = pl.multiple_of((g * 3 + c) * NPAD + r0, 128)
            pltpu.sync_copy(p_h.at[pl.ds(o, R2C)], bufs[g * 3 + c])
    i3 = lax.broadcasted_iota(jnp.int32, (L,), 0) * 3

    def vop(m, carry):
        w = pl.ds(m * L, L)
        x = p0x[w] + p1x[w]
        y = p0y[w] + p1y[w]
        z = p0z[w] + p1z[w]
        base = m * (3 * L) + i3
        plsc.store_scatter(stage, [base], x)
        plsc.store_scatter(stage, [base + 1], y)
        plsc.store_scatter(stage, [base + 2], z)
        return carry

    lax.fori_loop(0, R2C // L, vop, 0)
    pltpu.sync_copy(stage, out_h.at[pl.ds(pl.multiple_of(wid * W2C, 128), W2C)])


def kernel(dipoles, cell, positions, neighbor_indices, neighbor_vectors):
    del cell, positions
    idx = neighbor_indices.astype(jnp.int32).reshape(-1)
    vec = neighbor_vectors.astype(jnp.float32).reshape(-1)
    dip = jnp.concatenate(
        [dipoles.astype(jnp.float32).T,
         jnp.zeros((3, NPAD - N), jnp.float32)], axis=1).reshape(-1)
    zeros = jnp.zeros((NPAD,), jnp.float32)
    part = _edge_kernel(dip, zeros, idx, vec)
    flat = _combine_kernel(part)
    return flat[: N * 3].reshape(N, 3)
